# baseline (speedup 1.0000x reference)
import functools

import jax
import jax.numpy as jnp
from jax import lax
from jax.experimental import pallas as pl
from jax.experimental.pallas import tpu as pltpu
from jax.experimental.pallas import tpu_sc as plsc

N = 2048
E = 65536
D = 128
FIN = 64
H = 128
KNN = 10

NN = N * N
OUT_WORDS = 3 * NN
PAD_WORDS = 262144
PADDED = OUT_WORDS + PAD_WORDS
TRASH = OUT_WORDS
HALFW = OUT_WORDS // 2

EB1 = 4096
ES1 = E // EB1
EB3 = 512
ES3 = E // EB3
RB = 128
NRB = N // RB

SC_TOT = 2 * E + 4 * N * KNN
SC_PER_TILE = SC_TOT // 16
SC_CH = SC_PER_TILE // 128
SC_G = 8
ZW = 32768

def _dot(a, b, dims):
    return lax.dot_general(a, b, (dims, ((), ())),
                           precision=lax.Precision.DEFAULT,
                           preferred_element_type=jnp.float32)


def _estats_body(x_ref, o_ref):
    i = pl.program_id(0)
    x = x_ref[...]
    blk = jnp.concatenate([jnp.sum(x, axis=0, keepdims=True),
                           jnp.sum(x * x, axis=0, keepdims=True)], axis=0)

    @pl.when(i == 0)
    def _():
        o_ref[...] = blk

    @pl.when(i > 0)
    def _():
        o_ref[...] += blk


def _estats(edge_input):
    return pl.pallas_call(
        _estats_body,
        grid=(ES1,),
        in_specs=[pl.BlockSpec((EB1, D), lambda i: (i, 0))],
        out_specs=pl.BlockSpec((2, D), lambda i: (0, 0)),
        out_shape=jax.ShapeDtypeStruct((2, D), jnp.float32),
    )(edge_input)


def _standardize(x, st_ref):
    mu = st_ref[0:1, :] / E
    ex2 = st_ref[1:2, :] / E
    sig = jnp.sqrt(jnp.maximum(ex2 - mu * mu, 0.0)) + 1e-6
    return (x - mu) / sig


def _hstats_body(x_ref, st_ref, w1_ref, b1_ref, o_ref):
    i = pl.program_id(0)
    ei = _standardize(x_ref[...], st_ref)
    w1 = w1_ref[...]
    b1 = b1_ref[...]
    h1 = jnp.maximum(_dot(ei[:, :FIN], w1, ((1,), (0,))) + b1, 0.0)
    h2 = jnp.maximum(_dot(ei[:, FIN:], w1, ((1,), (0,))) + b1, 0.0)
    blk = jnp.concatenate([
        jnp.sum(h1, axis=0, keepdims=True),
        jnp.sum(h1 * h1, axis=0, keepdims=True),
        jnp.sum(h2, axis=0, keepdims=True),
        jnp.sum(h2 * h2, axis=0, keepdims=True),
    ], axis=0)

    @pl.when(i == 0)
    def _():
        o_ref[...] = blk

    @pl.when(i > 0)
    def _():
        o_ref[...] += blk


def _hstats(edge_input, stats, W1, b1):
    return pl.pallas_call(
        _hstats_body,
        grid=(ES1,),
        in_specs=[
            pl.BlockSpec((EB1, D), lambda i: (i, 0)),
            pl.BlockSpec((2, D), lambda i: (0, 0)),
            pl.BlockSpec((FIN, H), lambda i: (0, 0)),
            pl.BlockSpec((1, H), lambda i: (0, 0)),
        ],
        out_specs=pl.BlockSpec((4, D), lambda i: (0, 0)),
        out_shape=jax.ShapeDtypeStruct((4, D), jnp.float32),
    )(edge_input, stats, W1, b1)


def _edgeout_body(x_ref, st_ref, hs_ref, w1_ref, b1_ref, g_ref, be_ref,
                  w2_ref, b2_ref, ei_ref, w_ref, fsd_ref, fds_ref):
    ei = _standardize(x_ref[...], st_ref)
    w1 = w1_ref[...]
    b1 = b1_ref[...]
    gamma = g_ref[...]
    beta = be_ref[...]
    w2 = w2_ref[...]
    b2 = b2_ref[...]

    def tower(z, hs0, hs1):
        h = jnp.maximum(_dot(z, w1, ((1,), (0,))) + b1, 0.0)
        m = hs0 / E
        v = hs1 / E - m * m
        hn = (h - m) * (gamma / jnp.sqrt(v + 1e-5)) + beta
        return _dot(hn, w2, ((1,), (0,))) + b2

    o1 = tower(ei[:, :FIN], hs_ref[0:1, :], hs_ref[1:2, :])
    o2 = tower(ei[:, FIN:], hs_ref[2:3, :], hs_ref[3:4, :])
    n1 = jnp.maximum(jnp.sqrt(jnp.sum(o1 * o1, axis=1, keepdims=True)), 1e-8)
    n2 = jnp.maximum(jnp.sqrt(jnp.sum(o2 * o2, axis=1, keepdims=True)), 1e-8)
    cos = jnp.sum(o1 * o2, axis=1, keepdims=True) / (n1 * n2)
    w_ref[...] = ((cos + 1.0) * 0.5)[None]

    eb = ei_ref[...]
    src = eb[:, :, 0:1]
    dst = eb[:, :, 1:2]
    fsd_ref[...] = src * N + dst
    fds_ref[...] = dst * N + src


def _edgeout(edge_input, stats, hstats, W1, b1, gamma, beta, W2, b2, eiT):
    c = pl.pallas_call(
        _edgeout_body,
        grid=(ES3,),
        in_specs=[
            pl.BlockSpec((EB3, D), lambda i: (i, 0)),
            pl.BlockSpec((2, D), lambda i: (0, 0)),
            pl.BlockSpec((4, D), lambda i: (0, 0)),
            pl.BlockSpec((FIN, H), lambda i: (0, 0)),
            pl.BlockSpec((1, H), lambda i: (0, 0)),
            pl.BlockSpec((1, H), lambda i: (0, 0)),
            pl.BlockSpec((1, H), lambda i: (0, 0)),
            pl.BlockSpec((H, H), lambda i: (0, 0)),
            pl.BlockSpec((1, H), lambda i: (0, 0)),
            pl.BlockSpec((1, EB3, 2), lambda i: (i, 0, 0)),
        ],
        out_specs=[
            pl.BlockSpec((1, EB3, 1), lambda i: (i, 0, 0)),
            pl.BlockSpec((1, EB3, 1), lambda i: (i, 0, 0)),
            pl.BlockSpec((1, EB3, 1), lambda i: (i, 0, 0)),
        ],
        out_shape=[
            jax.ShapeDtypeStruct((ES3, EB3, 1), jnp.float32),
            jax.ShapeDtypeStruct((ES3, EB3, 1), jnp.int32),
            jax.ShapeDtypeStruct((ES3, EB3, 1), jnp.int32),
        ],
    )
    return c(edge_input, stats, hstats, W1, b1, gamma, beta, W2, b2, eiT)


def _topk_body(f_ref, fa_ref, fb_ref, normed):
    m = pl.program_id(0)
    i = pl.program_id(1)

    @pl.when(i == 0)
    def _():
        x = f_ref[0]
        normed[...] = x / jnp.sqrt(jnp.sum(x * x, axis=1, keepdims=True))

    nb = normed[pl.ds(i * RB, RB), :]
    s = _dot(nb, normed[...], ((1,), (1,)))
    s = (s + 1.0) * 0.5
    colidx = lax.broadcasted_iota(jnp.int32, (RB, N), 1)
    rowg = i * RB + lax.broadcasted_iota(jnp.int32, (RB, 1), 0)
    offm = (m + 1) * NN
    fa = []
    fb = []
    for _ in range(KNN):
        mx = jnp.max(s, axis=1, keepdims=True)
        cand = jnp.min(jnp.where(s >= mx, colidx, N), axis=1, keepdims=True)
        s = jnp.where(colidx == cand, -jnp.inf, s)
        fa.append(offm + rowg * N + cand)
        fb.append(offm + cand * N + rowg)
    fa_ref[0, 0] = jnp.concatenate(fa, axis=1)
    fb_ref[0, 0] = jnp.concatenate(fb, axis=1)


def _topk(feats):
    return pl.pallas_call(
        _topk_body,
        grid=(2, NRB),
        in_specs=[pl.BlockSpec((1, N, D), lambda m, i: (m, 0, 0))],
        out_specs=[
            pl.BlockSpec((1, 1, RB, KNN), lambda m, i: (m, i, 0, 0)),
            pl.BlockSpec((1, 1, RB, KNN), lambda m, i: (m, i, 0, 0)),
        ],
        out_shape=[
            jax.ShapeDtypeStruct((2, NRB, RB, KNN), jnp.int32),
            jax.ShapeDtypeStruct((2, NRB, RB, KNN), jnp.int32),
        ],
        scratch_shapes=[pltpu.VMEM((N, D), jnp.float32)],
    )(feats)


def _sc_scatter(idx_t, val_t):
    mesh = plsc.VectorSubcoreMesh(core_axis_name="c", subcore_axis_name="s")

    @functools.partial(
        pl.kernel,
        out_type=jax.ShapeDtypeStruct((PADDED,), jnp.float32),
        mesh=mesh,
        scratch_types=[
            pltpu.VMEM((SC_CH, 128), jnp.int32),
            pltpu.VMEM((SC_CH, 128), jnp.float32),
            pltpu.VMEM((ZW,), jnp.float32),
            pltpu.SemaphoreType.DMA,
            pltpu.SemaphoreType.DMA,
        ],
    )
    def scat(idx_hbm, val_hbm, out_hbm, idx_v, val_v, zbuf, sem, zsem):
        cid = lax.axis_index("c")
        sid = lax.axis_index("s")

        def zinit(j, carry):
            zbuf[pl.ds(j * 16, 16)] = jnp.zeros((16,), jnp.float32)
            return carry

        lax.fori_loop(0, ZW // 16, zinit, 0)
        stripe = HALFW // 16
        base = cid * HALFW + sid * stripe

        def zfire(j, carry):
            pltpu.async_copy(zbuf, out_hbm.at[pl.ds(base + j * ZW, ZW)], zsem)
            return carry

        lax.fori_loop(0, stripe // ZW, zfire, 0)

        pltpu.sync_copy(idx_hbm.at[sid], idx_v)
        pltpu.sync_copy(val_hbm.at[sid], val_v)

        lo = cid * HALFW
        hi = lo + HALFW
        tbase = TRASH + sid * (SC_CH * 128)

        def clamp(ci, carry):
            for l in range(8):
                v = idx_v[ci, pl.ds(l * 16, 16)]
                keep = (v >= lo) & (v < hi)
                trash = (tbase + ci * 128 + l * 16) + lax.iota(jnp.int32, 16)
                idx_v[ci, pl.ds(l * 16, 16)] = jnp.where(keep, v, trash)
            return carry

        lax.fori_loop(0, SC_CH, clamp, 0)

        def zdrain(j, carry):
            pltpu.make_async_copy(
                zbuf, out_hbm.at[pl.ds(base + j * ZW, ZW)], zsem).wait()
            return carry

        lax.fori_loop(0, stripe // ZW, zdrain, 0)
        plsc.subcore_barrier()

        def sloop(g, carry):
            cps = []
            for b in range(SC_G):
                ci = g * SC_G + b
                cps.append(pltpu.async_copy(
                    val_v.at[ci], out_hbm.at[idx_v.at[ci]], sem))
            for cp in cps:
                cp.wait()
            return carry

        lax.fori_loop(0, SC_CH // SC_G, sloop, 0)

    return scat(idx_t, val_t)


def _rownorm_body(a_ref, o_ref):
    i = pl.program_id(1)
    a = a_ref[0]
    rowg = i * RB + lax.broadcasted_iota(jnp.int32, (RB, 1), 0)
    col = lax.broadcasted_iota(jnp.int32, (RB, N), 1)
    a = a + jnp.where(col == rowg, 1.0, 0.0)
    s = jnp.sum(a, axis=1, keepdims=True)
    o_ref[0] = a * (1.0 / s)


def _rownorm(flat49):
    return pl.pallas_call(
        _rownorm_body,
        grid=(3, NRB),
        in_specs=[pl.BlockSpec((1, RB, N), lambda m, i: (m * NRB + i, 0, 0))],
        out_specs=pl.BlockSpec((1, RB, N), lambda m, i: (m * NRB + i, 0, 0)),
        out_shape=jax.ShapeDtypeStruct((3 * NRB, RB, N), jnp.float32),
    )(flat49)


def kernel(embeddings, t1_features, edge_input, W1, b1, gamma, beta, W2, b2,
           edge_index):
    b1r = b1.reshape(1, H)
    gr = gamma.reshape(1, H)
    ber = beta.reshape(1, H)
    b2r = b2.reshape(1, H)
    eiT = edge_index.T.astype(jnp.int32).reshape(ES3, EB3, 2)

    stats = _estats(edge_input)
    hstats = _hstats(edge_input, stats, W1, b1r)
    w3, fsd3, fds3 = _edgeout(edge_input, stats, hstats, W1, b1r, gr, ber,
                              W2, b2r, eiT)
    feats = jnp.stack([embeddings, t1_features])
    fa, fb = _topk(feats)

    wflat = w3.reshape(E)
    idx_all = jnp.concatenate([fsd3.reshape(E), fds3.reshape(E),
                               fa.reshape(2 * N * KNN), fb.reshape(2 * N * KNN)])
    val_all = jnp.concatenate([wflat, wflat,
                               jnp.ones((4 * N * KNN,), jnp.float32)])
    idx_t = idx_all.reshape(16, SC_CH, 128)
    val_t = val_all.reshape(16, SC_CH, 128)

    flat = _sc_scatter(idx_t, val_t)
    out = _rownorm(flat.reshape(PADDED // (RB * N), RB, N))
    return out.reshape(3, N, N)


# speedup vs baseline: 1.0015x; 1.0015x over previous
import functools

import jax
import jax.numpy as jnp
from jax import lax
from jax.experimental import pallas as pl
from jax.experimental.pallas import tpu as pltpu
from jax.experimental.pallas import tpu_sc as plsc

N = 2048
E = 65536
D = 128
FIN = 64
H = 128
KNN = 10

NN = N * N
OUT_WORDS = 3 * NN
PAD_WORDS = 262144
PADDED = OUT_WORDS + PAD_WORDS
TRASH = OUT_WORDS
HALFW = OUT_WORDS // 2

EB1 = 4096
ES1 = E // EB1
EB3 = 512
ES3 = E // EB3
RB = 128
NRB = N // RB

SC_TOT = 2 * E + 4 * N * KNN
SC_PER_TILE = SC_TOT // 16
SC_CH = SC_PER_TILE // 128
SC_G = 26
ZW = 32768

def _dot(a, b, dims):
    return lax.dot_general(a, b, (dims, ((), ())),
                           precision=lax.Precision.DEFAULT,
                           preferred_element_type=jnp.float32)


def _estats_body(x_ref, o_ref):
    i = pl.program_id(0)
    x = x_ref[...]
    blk = jnp.concatenate([jnp.sum(x, axis=0, keepdims=True),
                           jnp.sum(x * x, axis=0, keepdims=True)], axis=0)

    @pl.when(i == 0)
    def _():
        o_ref[...] = blk

    @pl.when(i > 0)
    def _():
        o_ref[...] += blk


def _estats(edge_input):
    return pl.pallas_call(
        _estats_body,
        grid=(ES1,),
        in_specs=[pl.BlockSpec((EB1, D), lambda i: (i, 0))],
        out_specs=pl.BlockSpec((2, D), lambda i: (0, 0)),
        out_shape=jax.ShapeDtypeStruct((2, D), jnp.float32),
    )(edge_input)


def _standardize(x, st_ref):
    mu = st_ref[0:1, :] / E
    ex2 = st_ref[1:2, :] / E
    sig = jnp.sqrt(jnp.maximum(ex2 - mu * mu, 0.0)) + 1e-6
    return (x - mu) / sig


def _hstats_body(x_ref, st_ref, w1_ref, b1_ref, o_ref):
    i = pl.program_id(0)
    ei = _standardize(x_ref[...], st_ref)
    w1 = w1_ref[...]
    b1 = b1_ref[...]
    h1 = jnp.maximum(_dot(ei[:, :FIN], w1, ((1,), (0,))) + b1, 0.0)
    h2 = jnp.maximum(_dot(ei[:, FIN:], w1, ((1,), (0,))) + b1, 0.0)
    blk = jnp.concatenate([
        jnp.sum(h1, axis=0, keepdims=True),
        jnp.sum(h1 * h1, axis=0, keepdims=True),
        jnp.sum(h2, axis=0, keepdims=True),
        jnp.sum(h2 * h2, axis=0, keepdims=True),
    ], axis=0)

    @pl.when(i == 0)
    def _():
        o_ref[...] = blk

    @pl.when(i > 0)
    def _():
        o_ref[...] += blk


def _hstats(edge_input, stats, W1, b1):
    return pl.pallas_call(
        _hstats_body,
        grid=(ES1,),
        in_specs=[
            pl.BlockSpec((EB1, D), lambda i: (i, 0)),
            pl.BlockSpec((2, D), lambda i: (0, 0)),
            pl.BlockSpec((FIN, H), lambda i: (0, 0)),
            pl.BlockSpec((1, H), lambda i: (0, 0)),
        ],
        out_specs=pl.BlockSpec((4, D), lambda i: (0, 0)),
        out_shape=jax.ShapeDtypeStruct((4, D), jnp.float32),
    )(edge_input, stats, W1, b1)


def _edgeout_body(x_ref, st_ref, hs_ref, w1_ref, b1_ref, g_ref, be_ref,
                  w2_ref, b2_ref, ei_ref, w_ref, fsd_ref, fds_ref):
    ei = _standardize(x_ref[...], st_ref)
    w1 = w1_ref[...]
    b1 = b1_ref[...]
    gamma = g_ref[...]
    beta = be_ref[...]
    w2 = w2_ref[...]
    b2 = b2_ref[...]

    def tower(z, hs0, hs1):
        h = jnp.maximum(_dot(z, w1, ((1,), (0,))) + b1, 0.0)
        m = hs0 / E
        v = hs1 / E - m * m
        hn = (h - m) * (gamma / jnp.sqrt(v + 1e-5)) + beta
        return _dot(hn, w2, ((1,), (0,))) + b2

    o1 = tower(ei[:, :FIN], hs_ref[0:1, :], hs_ref[1:2, :])
    o2 = tower(ei[:, FIN:], hs_ref[2:3, :], hs_ref[3:4, :])
    n1 = jnp.maximum(jnp.sqrt(jnp.sum(o1 * o1, axis=1, keepdims=True)), 1e-8)
    n2 = jnp.maximum(jnp.sqrt(jnp.sum(o2 * o2, axis=1, keepdims=True)), 1e-8)
    cos = jnp.sum(o1 * o2, axis=1, keepdims=True) / (n1 * n2)
    w_ref[...] = ((cos + 1.0) * 0.5)[None]

    eb = ei_ref[...]
    src = eb[:, :, 0:1]
    dst = eb[:, :, 1:2]
    fsd_ref[...] = src * N + dst
    fds_ref[...] = dst * N + src


def _edgeout(edge_input, stats, hstats, W1, b1, gamma, beta, W2, b2, eiT):
    c = pl.pallas_call(
        _edgeout_body,
        grid=(ES3,),
        in_specs=[
            pl.BlockSpec((EB3, D), lambda i: (i, 0)),
            pl.BlockSpec((2, D), lambda i: (0, 0)),
            pl.BlockSpec((4, D), lambda i: (0, 0)),
            pl.BlockSpec((FIN, H), lambda i: (0, 0)),
            pl.BlockSpec((1, H), lambda i: (0, 0)),
            pl.BlockSpec((1, H), lambda i: (0, 0)),
            pl.BlockSpec((1, H), lambda i: (0, 0)),
            pl.BlockSpec((H, H), lambda i: (0, 0)),
            pl.BlockSpec((1, H), lambda i: (0, 0)),
            pl.BlockSpec((1, EB3, 2), lambda i: (i, 0, 0)),
        ],
        out_specs=[
            pl.BlockSpec((1, EB3, 1), lambda i: (i, 0, 0)),
            pl.BlockSpec((1, EB3, 1), lambda i: (i, 0, 0)),
            pl.BlockSpec((1, EB3, 1), lambda i: (i, 0, 0)),
        ],
        out_shape=[
            jax.ShapeDtypeStruct((ES3, EB3, 1), jnp.float32),
            jax.ShapeDtypeStruct((ES3, EB3, 1), jnp.int32),
            jax.ShapeDtypeStruct((ES3, EB3, 1), jnp.int32),
        ],
    )
    return c(edge_input, stats, hstats, W1, b1, gamma, beta, W2, b2, eiT)


def _topk_body(f_ref, fa_ref, fb_ref, normed):
    m = pl.program_id(0)
    i = pl.program_id(1)

    @pl.when(i == 0)
    def _():
        x = f_ref[0]
        normed[...] = x / jnp.sqrt(jnp.sum(x * x, axis=1, keepdims=True))

    nb = normed[pl.ds(i * RB, RB), :]
    s = _dot(nb, normed[...], ((1,), (1,)))
    s = (s + 1.0) * 0.5
    colidx = lax.broadcasted_iota(jnp.int32, (RB, N), 1)
    rowg = i * RB + lax.broadcasted_iota(jnp.int32, (RB, 1), 0)
    offm = (m + 1) * NN
    fa = []
    fb = []
    for _ in range(KNN):
        mx = jnp.max(s, axis=1, keepdims=True)
        cand = jnp.min(jnp.where(s >= mx, colidx, N), axis=1, keepdims=True)
        s = jnp.where(colidx == cand, -jnp.inf, s)
        fa.append(offm + rowg * N + cand)
        fb.append(offm + cand * N + rowg)
    fa_ref[0, 0] = jnp.concatenate(fa, axis=1)
    fb_ref[0, 0] = jnp.concatenate(fb, axis=1)


def _topk(feats):
    return pl.pallas_call(
        _topk_body,
        grid=(2, NRB),
        in_specs=[pl.BlockSpec((1, N, D), lambda m, i: (m, 0, 0))],
        out_specs=[
            pl.BlockSpec((1, 1, RB, KNN), lambda m, i: (m, i, 0, 0)),
            pl.BlockSpec((1, 1, RB, KNN), lambda m, i: (m, i, 0, 0)),
        ],
        out_shape=[
            jax.ShapeDtypeStruct((2, NRB, RB, KNN), jnp.int32),
            jax.ShapeDtypeStruct((2, NRB, RB, KNN), jnp.int32),
        ],
        scratch_shapes=[pltpu.VMEM((N, D), jnp.float32)],
    )(feats)


def _sc_scatter(idx_t, val_t):
    mesh = plsc.VectorSubcoreMesh(core_axis_name="c", subcore_axis_name="s")

    @functools.partial(
        pl.kernel,
        out_type=jax.ShapeDtypeStruct((PADDED,), jnp.float32),
        mesh=mesh,
        scratch_types=[
            pltpu.VMEM((SC_CH, 128), jnp.int32),
            pltpu.VMEM((SC_CH, 128), jnp.float32),
            pltpu.VMEM((ZW,), jnp.float32),
            pltpu.SemaphoreType.DMA,
            pltpu.SemaphoreType.DMA,
        ],
    )
    def scat(idx_hbm, val_hbm, out_hbm, idx_v, val_v, zbuf, sem, zsem):
        cid = lax.axis_index("c")
        sid = lax.axis_index("s")

        def zinit(j, carry):
            zbuf[pl.ds(j * 16, 16)] = jnp.zeros((16,), jnp.float32)
            return carry

        lax.fori_loop(0, ZW // 16, zinit, 0)
        stripe = HALFW // 16
        base = cid * HALFW + sid * stripe

        def zfire(j, carry):
            pltpu.async_copy(zbuf, out_hbm.at[pl.ds(base + j * ZW, ZW)], zsem)
            return carry

        lax.fori_loop(0, stripe // ZW, zfire, 0)

        pltpu.sync_copy(idx_hbm.at[sid], idx_v)
        pltpu.sync_copy(val_hbm.at[sid], val_v)

        lo = cid * HALFW
        hi = lo + HALFW
        tbase = TRASH + sid * (SC_CH * 128)

        def clamp(ci, carry):
            for l in range(8):
                v = idx_v[ci, pl.ds(l * 16, 16)]
                keep = (v >= lo) & (v < hi)
                trash = (tbase + ci * 128 + l * 16) + lax.iota(jnp.int32, 16)
                idx_v[ci, pl.ds(l * 16, 16)] = jnp.where(keep, v, trash)
            return carry

        lax.fori_loop(0, SC_CH, clamp, 0)

        def zdrain(j, carry):
            pltpu.make_async_copy(
                zbuf, out_hbm.at[pl.ds(base + j * ZW, ZW)], zsem).wait()
            return carry

        lax.fori_loop(0, stripe // ZW, zdrain, 0)
        plsc.subcore_barrier()

        def sloop(g, carry):
            cps = []
            for b in range(SC_G):
                ci = g * SC_G + b
                cps.append(pltpu.async_copy(
                    val_v.at[ci], out_hbm.at[idx_v.at[ci]], sem))
            for cp in cps:
                cp.wait()
            return carry

        lax.fori_loop(0, SC_CH // SC_G, sloop, 0)

    return scat(idx_t, val_t)


def _rownorm_body(a_ref, o_ref):
    i = pl.program_id(1)
    a = a_ref[0]
    rowg = i * RB + lax.broadcasted_iota(jnp.int32, (RB, 1), 0)
    col = lax.broadcasted_iota(jnp.int32, (RB, N), 1)
    a = a + jnp.where(col == rowg, 1.0, 0.0)
    s = jnp.sum(a, axis=1, keepdims=True)
    o_ref[0] = a * (1.0 / s)


def _rownorm(flat49):
    return pl.pallas_call(
        _rownorm_body,
        grid=(3, NRB),
        in_specs=[pl.BlockSpec((1, RB, N), lambda m, i: (m * NRB + i, 0, 0))],
        out_specs=pl.BlockSpec((1, RB, N), lambda m, i: (m * NRB + i, 0, 0)),
        out_shape=jax.ShapeDtypeStruct((3 * NRB, RB, N), jnp.float32),
    )(flat49)


def kernel(embeddings, t1_features, edge_input, W1, b1, gamma, beta, W2, b2,
           edge_index):
    b1r = b1.reshape(1, H)
    gr = gamma.reshape(1, H)
    ber = beta.reshape(1, H)
    b2r = b2.reshape(1, H)
    eiT = edge_index.T.astype(jnp.int32).reshape(ES3, EB3, 2)

    stats = _estats(edge_input)
    hstats = _hstats(edge_input, stats, W1, b1r)
    w3, fsd3, fds3 = _edgeout(edge_input, stats, hstats, W1, b1r, gr, ber,
                              W2, b2r, eiT)
    feats = jnp.stack([embeddings, t1_features])
    fa, fb = _topk(feats)

    wflat = w3.reshape(E)
    idx_all = jnp.concatenate([fsd3.reshape(E), fds3.reshape(E),
                               fa.reshape(2 * N * KNN), fb.reshape(2 * N * KNN)])
    val_all = jnp.concatenate([wflat, wflat,
                               jnp.ones((4 * N * KNN,), jnp.float32)])
    idx_t = idx_all.reshape(16, SC_CH, 128)
    val_t = val_all.reshape(16, SC_CH, 128)

    flat = _sc_scatter(idx_t, val_t)
    out = _rownorm(flat.reshape(PADDED // (RB * N), RB, N))
    return out.reshape(3, N, N)


# speedup vs baseline: 1.5814x; 1.5791x over previous
import functools

import jax
import jax.numpy as jnp
from jax import lax
from jax.experimental import pallas as pl
from jax.experimental.pallas import tpu as pltpu
from jax.experimental.pallas import tpu_sc as plsc

N = 2048
E = 65536
D = 128
FIN = 64
H = 128
KNN = 10

NN = N * N
PAD_WORDS = 262144
PADDED = NN + PAD_WORDS
TRASH = NN
HALFW = NN // 2

EB1 = 4096
ES1 = E // EB1
EB3 = 512
ES3 = E // EB3
RB = 128
NRB = N // RB

SC_TOT = 2 * E
SC_PER_TILE = SC_TOT // 16
SC_CH = SC_PER_TILE // 128
SC_G = 8
ZW = 32768


def _dot(a, b, dims):
    return lax.dot_general(a, b, (dims, ((), ())),
                           precision=lax.Precision.DEFAULT,
                           preferred_element_type=jnp.float32)


def _estats_body(x_ref, o_ref):
    i = pl.program_id(0)
    x = x_ref[...]
    blk = jnp.concatenate([jnp.sum(x, axis=0, keepdims=True),
                           jnp.sum(x * x, axis=0, keepdims=True)], axis=0)

    @pl.when(i == 0)
    def _():
        o_ref[...] = blk

    @pl.when(i > 0)
    def _():
        o_ref[...] += blk


def _estats(edge_input):
    return pl.pallas_call(
        _estats_body,
        grid=(ES1,),
        in_specs=[pl.BlockSpec((EB1, D), lambda i: (i, 0))],
        out_specs=pl.BlockSpec((2, D), lambda i: (0, 0)),
        out_shape=jax.ShapeDtypeStruct((2, D), jnp.float32),
    )(edge_input)


def _standardize(x, st_ref):
    mu = st_ref[0:1, :] / E
    ex2 = st_ref[1:2, :] / E
    sig = jnp.sqrt(jnp.maximum(ex2 - mu * mu, 0.0)) + 1e-6
    return (x - mu) / sig


def _hstats_body(x_ref, st_ref, w1_ref, b1_ref, o_ref):
    i = pl.program_id(0)
    ei = _standardize(x_ref[...], st_ref)
    w1 = w1_ref[...]
    b1 = b1_ref[...]
    h1 = jnp.maximum(_dot(ei[:, :FIN], w1, ((1,), (0,))) + b1, 0.0)
    h2 = jnp.maximum(_dot(ei[:, FIN:], w1, ((1,), (0,))) + b1, 0.0)
    blk = jnp.concatenate([
        jnp.sum(h1, axis=0, keepdims=True),
        jnp.sum(h1 * h1, axis=0, keepdims=True),
        jnp.sum(h2, axis=0, keepdims=True),
        jnp.sum(h2 * h2, axis=0, keepdims=True),
    ], axis=0)

    @pl.when(i == 0)
    def _():
        o_ref[...] = blk

    @pl.when(i > 0)
    def _():
        o_ref[...] += blk


def _hstats(edge_input, stats, W1, b1):
    return pl.pallas_call(
        _hstats_body,
        grid=(ES1,),
        in_specs=[
            pl.BlockSpec((EB1, D), lambda i: (i, 0)),
            pl.BlockSpec((2, D), lambda i: (0, 0)),
            pl.BlockSpec((FIN, H), lambda i: (0, 0)),
            pl.BlockSpec((1, H), lambda i: (0, 0)),
        ],
        out_specs=pl.BlockSpec((4, D), lambda i: (0, 0)),
        out_shape=jax.ShapeDtypeStruct((4, D), jnp.float32),
    )(edge_input, stats, W1, b1)


def _edgeout_body(x_ref, st_ref, hs_ref, w1_ref, b1_ref, g_ref, be_ref,
                  w2_ref, b2_ref, ei_ref, w_ref, fsd_ref, fds_ref):
    ei = _standardize(x_ref[...], st_ref)
    w1 = w1_ref[...]
    b1 = b1_ref[...]
    gamma = g_ref[...]
    beta = be_ref[...]
    w2 = w2_ref[...]
    b2 = b2_ref[...]

    def tower(z, hs0, hs1):
        h = jnp.maximum(_dot(z, w1, ((1,), (0,))) + b1, 0.0)
        m = hs0 / E
        v = hs1 / E - m * m
        hn = (h - m) * (gamma / jnp.sqrt(v + 1e-5)) + beta
        return _dot(hn, w2, ((1,), (0,))) + b2

    o1 = tower(ei[:, :FIN], hs_ref[0:1, :], hs_ref[1:2, :])
    o2 = tower(ei[:, FIN:], hs_ref[2:3, :], hs_ref[3:4, :])
    n1 = jnp.maximum(jnp.sqrt(jnp.sum(o1 * o1, axis=1, keepdims=True)), 1e-8)
    n2 = jnp.maximum(jnp.sqrt(jnp.sum(o2 * o2, axis=1, keepdims=True)), 1e-8)
    cos = jnp.sum(o1 * o2, axis=1, keepdims=True) / (n1 * n2)
    w_ref[...] = ((cos + 1.0) * 0.5)[None]

    eb = ei_ref[...]
    src = eb[:, :, 0:1]
    dst = eb[:, :, 1:2]
    fsd_ref[...] = src * N + dst
    fds_ref[...] = dst * N + src


def _edgeout(edge_input, stats, hstats, W1, b1, gamma, beta, W2, b2, eiT):
    c = pl.pallas_call(
        _edgeout_body,
        grid=(ES3,),
        in_specs=[
            pl.BlockSpec((EB3, D), lambda i: (i, 0)),
            pl.BlockSpec((2, D), lambda i: (0, 0)),
            pl.BlockSpec((4, D), lambda i: (0, 0)),
            pl.BlockSpec((FIN, H), lambda i: (0, 0)),
            pl.BlockSpec((1, H), lambda i: (0, 0)),
            pl.BlockSpec((1, H), lambda i: (0, 0)),
            pl.BlockSpec((1, H), lambda i: (0, 0)),
            pl.BlockSpec((H, H), lambda i: (0, 0)),
            pl.BlockSpec((1, H), lambda i: (0, 0)),
            pl.BlockSpec((1, EB3, 2), lambda i: (i, 0, 0)),
        ],
        out_specs=[
            pl.BlockSpec((1, EB3, 1), lambda i: (i, 0, 0)),
            pl.BlockSpec((1, EB3, 1), lambda i: (i, 0, 0)),
            pl.BlockSpec((1, EB3, 1), lambda i: (i, 0, 0)),
        ],
        out_shape=[
            jax.ShapeDtypeStruct((ES3, EB3, 1), jnp.float32),
            jax.ShapeDtypeStruct((ES3, EB3, 1), jnp.int32),
            jax.ShapeDtypeStruct((ES3, EB3, 1), jnp.int32),
        ],
    )
    return c(edge_input, stats, hstats, W1, b1, gamma, beta, W2, b2, eiT)


def _topk_body(f_ref, a_ref, at_ref, normed):
    i = pl.program_id(1)

    @pl.when(i == 0)
    def _():
        x = f_ref[0]
        normed[...] = x / jnp.sqrt(jnp.sum(x * x, axis=1, keepdims=True))

    nb = normed[pl.ds(i * RB, RB), :]
    s = _dot(nb, normed[...], ((1,), (1,)))
    s = (s + 1.0) * 0.5
    colidx = lax.broadcasted_iota(jnp.int32, (RB, N), 1)
    for _ in range(KNN):
        mx = jnp.max(s, axis=1, keepdims=True)
        cand = jnp.min(jnp.where(s >= mx, colidx, N), axis=1, keepdims=True)
        s = jnp.where(colidx == cand, -jnp.inf, s)
    ablk = jnp.where(s == -jnp.inf, 1.0, 0.0)
    a_ref[0] = ablk
    at_ref[0] = ablk.T


def _topk(feats):
    return pl.pallas_call(
        _topk_body,
        grid=(2, NRB),
        in_specs=[pl.BlockSpec((1, N, D), lambda m, i: (m, 0, 0))],
        out_specs=[
            pl.BlockSpec((1, RB, N), lambda m, i: (m, i, 0)),
            pl.BlockSpec((1, N, RB), lambda m, i: (m, 0, i)),
        ],
        out_shape=[
            jax.ShapeDtypeStruct((2, N, N), jnp.float32),
            jax.ShapeDtypeStruct((2, N, N), jnp.float32),
        ],
        scratch_shapes=[pltpu.VMEM((N, D), jnp.float32)],
    )(feats)


def _sc_scatter(idx_t, val_t):
    mesh = plsc.VectorSubcoreMesh(core_axis_name="c", subcore_axis_name="s")

    @functools.partial(
        pl.kernel,
        out_type=jax.ShapeDtypeStruct((PADDED,), jnp.float32),
        mesh=mesh,
        scratch_types=[
            pltpu.VMEM((SC_CH, 128), jnp.int32),
            pltpu.VMEM((SC_CH, 128), jnp.float32),
            pltpu.VMEM((ZW,), jnp.float32),
            pltpu.SemaphoreType.DMA,
            pltpu.SemaphoreType.DMA,
        ],
    )
    def scat(idx_hbm, val_hbm, out_hbm, idx_v, val_v, zbuf, sem, zsem):
        cid = lax.axis_index("c")
        sid = lax.axis_index("s")

        def zinit(j, carry):
            zbuf[pl.ds(j * 16, 16)] = jnp.zeros((16,), jnp.float32)
            return carry

        lax.fori_loop(0, ZW // 16, zinit, 0)
        stripe = HALFW // 16
        base = cid * HALFW + sid * stripe

        def zfire(j, carry):
            pltpu.async_copy(zbuf, out_hbm.at[pl.ds(base + j * ZW, ZW)], zsem)
            return carry

        lax.fori_loop(0, stripe // ZW, zfire, 0)

        pltpu.sync_copy(idx_hbm.at[sid], idx_v)
        pltpu.sync_copy(val_hbm.at[sid], val_v)

        lo = cid * HALFW
        hi = lo + HALFW
        tbase = TRASH + sid * (SC_CH * 128)

        def clamp(ci, carry):
            for l in range(8):
                v = idx_v[ci, pl.ds(l * 16, 16)]
                keep = (v >= lo) & (v < hi)
                trash = (tbase + ci * 128 + l * 16) + lax.iota(jnp.int32, 16)
                idx_v[ci, pl.ds(l * 16, 16)] = jnp.where(keep, v, trash)
            return carry

        lax.fori_loop(0, SC_CH, clamp, 0)

        def zdrain(j, carry):
            pltpu.make_async_copy(
                zbuf, out_hbm.at[pl.ds(base + j * ZW, ZW)], zsem).wait()
            return carry

        lax.fori_loop(0, stripe // ZW, zdrain, 0)
        plsc.subcore_barrier()

        def sloop(g, carry):
            cps = []
            for b in range(SC_G):
                ci = g * SC_G + b
                cps.append(pltpu.async_copy(
                    val_v.at[ci], out_hbm.at[idx_v.at[ci]], sem))
            for cp in cps:
                cp.wait()
            return carry

        lax.fori_loop(0, SC_CH // SC_G, sloop, 0)

    return scat(idx_t, val_t)


def _rownorm_body(sadj_ref, a_ref, at_ref, o_ref):
    m = pl.program_id(0)
    i = pl.program_id(1)
    knn = jnp.maximum(a_ref[0], at_ref[0])
    a = jnp.where(m == 0, sadj_ref[0], knn)
    rowg = i * RB + lax.broadcasted_iota(jnp.int32, (RB, 1), 0)
    col = lax.broadcasted_iota(jnp.int32, (RB, N), 1)
    a = a + jnp.where(col == rowg, 1.0, 0.0)
    s = jnp.sum(a, axis=1, keepdims=True)
    o_ref[0] = a * (1.0 / s)


def _rownorm(sadj17, knn_a, knn_at):
    return pl.pallas_call(
        _rownorm_body,
        grid=(3, NRB),
        in_specs=[
            pl.BlockSpec((1, RB, N),
                         lambda m, i: ((m == 0) * i, 0, 0)),
            pl.BlockSpec((1, RB, N),
                         lambda m, i: ((m > 0) * (m - 1), (m > 0) * i, 0)),
            pl.BlockSpec((1, RB, N),
                         lambda m, i: ((m > 0) * (m - 1), (m > 0) * i, 0)),
        ],
        out_specs=pl.BlockSpec((1, RB, N), lambda m, i: (m, i, 0)),
        out_shape=jax.ShapeDtypeStruct((3, N, N), jnp.float32),
    )(sadj17, knn_a, knn_at)


def kernel(embeddings, t1_features, edge_input, W1, b1, gamma, beta, W2, b2,
           edge_index):
    b1r = b1.reshape(1, H)
    gr = gamma.reshape(1, H)
    ber = beta.reshape(1, H)
    b2r = b2.reshape(1, H)
    eiT = edge_index.T.astype(jnp.int32).reshape(ES3, EB3, 2)

    stats = _estats(edge_input)
    hstats = _hstats(edge_input, stats, W1, b1r)
    w3, fsd3, fds3 = _edgeout(edge_input, stats, hstats, W1, b1r, gr, ber,
                              W2, b2r, eiT)

    wflat = w3.reshape(E)
    idx_t = jnp.concatenate([fsd3.reshape(E),
                             fds3.reshape(E)]).reshape(16, SC_CH, 128)
    val_t = jnp.concatenate([wflat, wflat]).reshape(16, SC_CH, 128)
    sadj_flat = _sc_scatter(idx_t, val_t)

    feats = jnp.stack([embeddings, t1_features])
    knn_a, knn_at = _topk(feats)

    sadj17 = sadj_flat.reshape(PADDED // (RB * N), RB, N)
    return _rownorm(sadj17, knn_a, knn_at)


# speedup vs baseline: 2.7252x; 1.7233x over previous
import functools

import jax
import jax.numpy as jnp
from jax import lax
from jax.experimental import pallas as pl
from jax.experimental.pallas import tpu as pltpu
from jax.experimental.pallas import tpu_sc as plsc

N = 2048
E = 65536
D = 128
FIN = 64
H = 128
KNN = 10

NN = N * N
QW = NN // 4
TW = 131072
SPW = QW + TW

EB1 = 4096
ES1 = E // EB1
EB3 = 512
ES3 = E // EB3
RB = 128
NRB = N // RB

SC_TOT = 2 * E
SC_PER_TILE = SC_TOT // 16
SC_CH = SC_PER_TILE // 128
SC_G = 8
ZW = 32768


def _dot(a, b, dims):
    return lax.dot_general(a, b, (dims, ((), ())),
                           precision=lax.Precision.DEFAULT,
                           preferred_element_type=jnp.float32)


def _estats_body(x_ref, o_ref):
    i = pl.program_id(0)
    x = x_ref[...]
    blk = jnp.concatenate([jnp.sum(x, axis=0, keepdims=True),
                           jnp.sum(x * x, axis=0, keepdims=True)], axis=0)

    @pl.when(i == 0)
    def _():
        o_ref[...] = blk

    @pl.when(i > 0)
    def _():
        o_ref[...] += blk


def _estats(edge_input):
    return pl.pallas_call(
        _estats_body,
        grid=(ES1,),
        in_specs=[pl.BlockSpec((EB1, D), lambda i: (i, 0))],
        out_specs=pl.BlockSpec((2, D), lambda i: (0, 0)),
        out_shape=jax.ShapeDtypeStruct((2, D), jnp.float32),
    )(edge_input)


def _standardize(x, st_ref):
    mu = st_ref[0:1, :] / E
    ex2 = st_ref[1:2, :] / E
    sig = jnp.sqrt(jnp.maximum(ex2 - mu * mu, 0.0)) + 1e-6
    return (x - mu) / sig


def _hstats_body(x_ref, st_ref, w1_ref, b1_ref, o_ref):
    i = pl.program_id(0)
    ei = _standardize(x_ref[...], st_ref)
    w1 = w1_ref[...]
    b1 = b1_ref[...]
    h1 = jnp.maximum(_dot(ei[:, :FIN], w1, ((1,), (0,))) + b1, 0.0)
    h2 = jnp.maximum(_dot(ei[:, FIN:], w1, ((1,), (0,))) + b1, 0.0)
    blk = jnp.concatenate([
        jnp.sum(h1, axis=0, keepdims=True),
        jnp.sum(h1 * h1, axis=0, keepdims=True),
        jnp.sum(h2, axis=0, keepdims=True),
        jnp.sum(h2 * h2, axis=0, keepdims=True),
    ], axis=0)

    @pl.when(i == 0)
    def _():
        o_ref[...] = blk

    @pl.when(i > 0)
    def _():
        o_ref[...] += blk


def _hstats(edge_input, stats, W1, b1):
    return pl.pallas_call(
        _hstats_body,
        grid=(ES1,),
        in_specs=[
            pl.BlockSpec((EB1, D), lambda i: (i, 0)),
            pl.BlockSpec((2, D), lambda i: (0, 0)),
            pl.BlockSpec((FIN, H), lambda i: (0, 0)),
            pl.BlockSpec((1, H), lambda i: (0, 0)),
        ],
        out_specs=pl.BlockSpec((4, D), lambda i: (0, 0)),
        out_shape=jax.ShapeDtypeStruct((4, D), jnp.float32),
    )(edge_input, stats, W1, b1)


def _edgeout_body(x_ref, st_ref, hs_ref, w1_ref, b1_ref, g_ref, be_ref,
                  w2_ref, b2_ref, ei_ref, w_ref, fsd_ref, fds_ref):
    ei = _standardize(x_ref[...], st_ref)
    w1 = w1_ref[...]
    b1 = b1_ref[...]
    gamma = g_ref[...]
    beta = be_ref[...]
    w2 = w2_ref[...]
    b2 = b2_ref[...]

    def tower(z, hs0, hs1):
        h = jnp.maximum(_dot(z, w1, ((1,), (0,))) + b1, 0.0)
        m = hs0 / E
        v = hs1 / E - m * m
        hn = (h - m) * (gamma / jnp.sqrt(v + 1e-5)) + beta
        return _dot(hn, w2, ((1,), (0,))) + b2

    o1 = tower(ei[:, :FIN], hs_ref[0:1, :], hs_ref[1:2, :])
    o2 = tower(ei[:, FIN:], hs_ref[2:3, :], hs_ref[3:4, :])
    n1 = jnp.maximum(jnp.sqrt(jnp.sum(o1 * o1, axis=1, keepdims=True)), 1e-8)
    n2 = jnp.maximum(jnp.sqrt(jnp.sum(o2 * o2, axis=1, keepdims=True)), 1e-8)
    cos = jnp.sum(o1 * o2, axis=1, keepdims=True) / (n1 * n2)
    w_ref[...] = ((cos + 1.0) * 0.5)[None]

    eb = ei_ref[...]
    src = eb[:, :, 0:1]
    dst = eb[:, :, 1:2]
    fsd_ref[...] = src * N + dst
    fds_ref[...] = dst * N + src


def _edgeout(edge_input, stats, hstats, W1, b1, gamma, beta, W2, b2, eiT):
    c = pl.pallas_call(
        _edgeout_body,
        grid=(ES3,),
        in_specs=[
            pl.BlockSpec((EB3, D), lambda i: (i, 0)),
            pl.BlockSpec((2, D), lambda i: (0, 0)),
            pl.BlockSpec((4, D), lambda i: (0, 0)),
            pl.BlockSpec((FIN, H), lambda i: (0, 0)),
            pl.BlockSpec((1, H), lambda i: (0, 0)),
            pl.BlockSpec((1, H), lambda i: (0, 0)),
            pl.BlockSpec((1, H), lambda i: (0, 0)),
            pl.BlockSpec((H, H), lambda i: (0, 0)),
            pl.BlockSpec((1, H), lambda i: (0, 0)),
            pl.BlockSpec((1, EB3, 2), lambda i: (i, 0, 0)),
        ],
        out_specs=[
            pl.BlockSpec((1, EB3, 1), lambda i: (i, 0, 0)),
            pl.BlockSpec((1, EB3, 1), lambda i: (i, 0, 0)),
            pl.BlockSpec((1, EB3, 1), lambda i: (i, 0, 0)),
        ],
        out_shape=[
            jax.ShapeDtypeStruct((ES3, EB3, 1), jnp.float32),
            jax.ShapeDtypeStruct((ES3, EB3, 1), jnp.int32),
            jax.ShapeDtypeStruct((ES3, EB3, 1), jnp.int32),
        ],
    )
    return c(edge_input, stats, hstats, W1, b1, gamma, beta, W2, b2, eiT)


def _topk_body(f_ref, a_ref, at_ref, normed):
    i = pl.program_id(1)

    @pl.when(i == 0)
    def _():
        x = f_ref[0]
        normed[...] = x / jnp.sqrt(jnp.sum(x * x, axis=1, keepdims=True))

    nb = normed[pl.ds(i * RB, RB), :]
    s = _dot(nb, normed[...], ((1,), (1,)))
    s = (s + 1.0) * 0.5
    colidx = lax.broadcasted_iota(jnp.int32, (RB, N), 1)
    for _ in range(KNN):
        mx = jnp.max(s, axis=1, keepdims=True)
        cand = jnp.min(jnp.where(s >= mx, colidx, N), axis=1, keepdims=True)
        s = jnp.where(colidx == cand, -jnp.inf, s)
    ablk = jnp.where(s == -jnp.inf, 1.0, 0.0)
    a_ref[0] = ablk
    at_ref[0] = ablk.T


def _topk(feats):
    return pl.pallas_call(
        _topk_body,
        grid=(2, NRB),
        in_specs=[pl.BlockSpec((1, N, D), lambda m, i: (m, 0, 0))],
        out_specs=[
            pl.BlockSpec((1, RB, N), lambda m, i: (m, i, 0)),
            pl.BlockSpec((1, N, RB), lambda m, i: (m, 0, i)),
        ],
        out_shape=[
            jax.ShapeDtypeStruct((2, N, N), jnp.float32),
            jax.ShapeDtypeStruct((2, N, N), jnp.float32),
        ],
        scratch_shapes=[pltpu.VMEM((N, D), jnp.float32)],
    )(feats)


def _sc_scatter(idx_t, val_t):
    mesh = plsc.VectorSubcoreMesh(core_axis_name="c", subcore_axis_name="s")

    @functools.partial(
        pl.kernel,
        out_type=jax.ShapeDtypeStruct((NN,), jnp.float32),
        mesh=mesh,
        scratch_types=[
            pltpu.VMEM((SC_CH, 128), jnp.int32),
            pltpu.VMEM((SC_CH, 128), jnp.float32),
            pltpu.VMEM((SC_CH, 128), jnp.int32),
            pltpu.VMEM((ZW,), jnp.float32),
            pltpu.VMEM_SHARED((SPW,), jnp.float32),
            pltpu.SemaphoreType.DMA,
            pltpu.SemaphoreType.DMA,
        ],
    )
    def scat(idx_hbm, val_hbm, out_hbm, idx_v, val_v, sidx_v, zbuf, sp, sem,
             zsem):
        cid = lax.axis_index("c")
        sid = lax.axis_index("s")

        pltpu.sync_copy(idx_hbm.at[sid], idx_v)
        pltpu.sync_copy(val_hbm.at[sid], val_v)

        def zinit(j, carry):
            zbuf[pl.ds(j * 16, 16)] = jnp.zeros((16,), jnp.float32)
            return carry

        lax.fori_loop(0, ZW // 16, zinit, 0)

        stripe = QW // 16
        tbase = QW + sid * (SC_CH * 128)

        for p in range(2):
            base = (2 * cid + p) * QW

            for j in range(stripe // ZW):
                pltpu.async_copy(
                    zbuf, sp.at[pl.ds(sid * stripe + j * ZW, ZW)], zsem)

            def clamp(ci, carry):
                for l in range(8):
                    v = idx_v[ci, pl.ds(l * 16, 16)] - base
                    keep = (v >= 0) & (v < QW)
                    trash = (tbase + ci * 128 + l * 16) + lax.iota(
                        jnp.int32, 16)
                    sidx_v[ci, pl.ds(l * 16, 16)] = jnp.where(keep, v, trash)
                return carry

            lax.fori_loop(0, SC_CH, clamp, 0)

            for j in range(stripe // ZW):
                pltpu.make_async_copy(
                    zbuf, sp.at[pl.ds(sid * stripe + j * ZW, ZW)],
                    zsem).wait()
            plsc.subcore_barrier()

            def sloop(g, carry):
                cps = []
                for b in range(SC_G):
                    ci = g * SC_G + b
                    cps.append(pltpu.async_copy(
                        val_v.at[ci], sp.at[sidx_v.at[ci]], sem))
                for cp in cps:
                    cp.wait()
                return carry

            lax.fori_loop(0, SC_CH // SC_G, sloop, 0)
            plsc.subcore_barrier()

            for j in range(stripe // ZW):
                pltpu.async_copy(
                    sp.at[pl.ds(sid * stripe + j * ZW, ZW)],
                    out_hbm.at[pl.ds(base + sid * stripe + j * ZW, ZW)],
                    zsem)
            for j in range(stripe // ZW):
                pltpu.make_async_copy(
                    sp.at[pl.ds(sid * stripe + j * ZW, ZW)],
                    out_hbm.at[pl.ds(base + sid * stripe + j * ZW, ZW)],
                    zsem).wait()

    return scat(idx_t, val_t)


def _rownorm_body(sadj_ref, a_ref, at_ref, o_ref):
    m = pl.program_id(0)
    i = pl.program_id(1)
    knn = jnp.maximum(a_ref[0], at_ref[0])
    a = jnp.where(m == 0, sadj_ref[0], knn)
    rowg = i * RB + lax.broadcasted_iota(jnp.int32, (RB, 1), 0)
    col = lax.broadcasted_iota(jnp.int32, (RB, N), 1)
    a = a + jnp.where(col == rowg, 1.0, 0.0)
    s = jnp.sum(a, axis=1, keepdims=True)
    o_ref[0] = a * (1.0 / s)


def _rownorm(sadj17, knn_a, knn_at):
    return pl.pallas_call(
        _rownorm_body,
        grid=(3, NRB),
        in_specs=[
            pl.BlockSpec((1, RB, N),
                         lambda m, i: ((m == 0) * i, 0, 0)),
            pl.BlockSpec((1, RB, N),
                         lambda m, i: ((m > 0) * (m - 1), (m > 0) * i, 0)),
            pl.BlockSpec((1, RB, N),
                         lambda m, i: ((m > 0) * (m - 1), (m > 0) * i, 0)),
        ],
        out_specs=pl.BlockSpec((1, RB, N), lambda m, i: (m, i, 0)),
        out_shape=jax.ShapeDtypeStruct((3, N, N), jnp.float32),
    )(sadj17, knn_a, knn_at)


def kernel(embeddings, t1_features, edge_input, W1, b1, gamma, beta, W2, b2,
           edge_index):
    b1r = b1.reshape(1, H)
    gr = gamma.reshape(1, H)
    ber = beta.reshape(1, H)
    b2r = b2.reshape(1, H)
    eiT = edge_index.T.astype(jnp.int32).reshape(ES3, EB3, 2)

    stats = _estats(edge_input)
    hstats = _hstats(edge_input, stats, W1, b1r)
    w3, fsd3, fds3 = _edgeout(edge_input, stats, hstats, W1, b1r, gr, ber,
                              W2, b2r, eiT)

    wflat = w3.reshape(E)
    idx_t = jnp.concatenate([fsd3.reshape(E),
                             fds3.reshape(E)]).reshape(16, SC_CH, 128)
    val_t = jnp.concatenate([wflat, wflat]).reshape(16, SC_CH, 128)
    sadj_flat = _sc_scatter(idx_t, val_t)

    feats = jnp.stack([embeddings, t1_features])
    knn_a, knn_at = _topk(feats)

    sadj16 = sadj_flat.reshape(NN // (RB * N), RB, N)
    return _rownorm(sadj16, knn_a, knn_at)


# speedup vs baseline: 3.2843x; 1.2051x over previous
import functools

import jax
import jax.numpy as jnp
from jax import lax
from jax.experimental import pallas as pl
from jax.experimental.pallas import tpu as pltpu
from jax.experimental.pallas import tpu_sc as plsc

N = 2048
E = 65536
D = 128
FIN = 64
H = 128
KNN = 10

NN = N * N
QW = NN // 4
TW = 131072
SPW = QW + TW

EB1 = 4096
ES1 = E // EB1
EB3 = 2048
ES3 = E // EB3
RB = 128
NRB = N // RB

SC_TOT = 2 * E
SC_CH = E // 16 // 128
SC_G = 8
ZW = 32768


def _dot(a, b, dims):
    return lax.dot_general(a, b, (dims, ((), ())),
                           precision=lax.Precision.DEFAULT,
                           preferred_element_type=jnp.float32)


def _estats_body(x_ref, o_ref):
    i = pl.program_id(0)
    x = x_ref[...]
    blk = jnp.concatenate([jnp.sum(x, axis=0, keepdims=True),
                           jnp.sum(x * x, axis=0, keepdims=True)], axis=0)

    @pl.when(i == 0)
    def _():
        o_ref[...] = blk

    @pl.when(i > 0)
    def _():
        o_ref[...] += blk


def _estats(edge_input):
    return pl.pallas_call(
        _estats_body,
        grid=(ES1,),
        in_specs=[pl.BlockSpec((EB1, D), lambda i: (i, 0))],
        out_specs=pl.BlockSpec((2, D), lambda i: (0, 0)),
        out_shape=jax.ShapeDtypeStruct((2, D), jnp.float32),
    )(edge_input)


def _standardize(x, st_ref):
    mu = st_ref[0:1, :] / E
    ex2 = st_ref[1:2, :] / E
    sig = jnp.sqrt(jnp.maximum(ex2 - mu * mu, 0.0)) + 1e-6
    return (x - mu) / sig


def _hstats_body(x_ref, st_ref, w1_ref, b1_ref, o_ref):
    i = pl.program_id(0)
    ei = _standardize(x_ref[...], st_ref)
    w1 = w1_ref[...]
    b1 = b1_ref[...]
    h1 = jnp.maximum(_dot(ei[:, :FIN], w1, ((1,), (0,))) + b1, 0.0)
    h2 = jnp.maximum(_dot(ei[:, FIN:], w1, ((1,), (0,))) + b1, 0.0)
    blk = jnp.concatenate([
        jnp.sum(h1, axis=0, keepdims=True),
        jnp.sum(h1 * h1, axis=0, keepdims=True),
        jnp.sum(h2, axis=0, keepdims=True),
        jnp.sum(h2 * h2, axis=0, keepdims=True),
    ], axis=0)

    @pl.when(i == 0)
    def _():
        o_ref[...] = blk

    @pl.when(i > 0)
    def _():
        o_ref[...] += blk


def _hstats(edge_input, stats, W1, b1):
    return pl.pallas_call(
        _hstats_body,
        grid=(ES1,),
        in_specs=[
            pl.BlockSpec((EB1, D), lambda i: (i, 0)),
            pl.BlockSpec((2, D), lambda i: (0, 0)),
            pl.BlockSpec((FIN, H), lambda i: (0, 0)),
            pl.BlockSpec((1, H), lambda i: (0, 0)),
        ],
        out_specs=pl.BlockSpec((4, D), lambda i: (0, 0)),
        out_shape=jax.ShapeDtypeStruct((4, D), jnp.float32),
    )(edge_input, stats, W1, b1)


def _edgeout_body(x_ref, st_ref, hs_ref, w1_ref, b1_ref, g_ref, be_ref,
                  w2_ref, b2_ref, ei_ref, w_ref, fsd_ref, fds_ref):
    ei = _standardize(x_ref[...], st_ref)
    w1 = w1_ref[...]
    b1 = b1_ref[...]
    gamma = g_ref[...]
    beta = be_ref[...]
    w2 = w2_ref[...]
    b2 = b2_ref[...]

    def tower(z, hs0, hs1):
        h = jnp.maximum(_dot(z, w1, ((1,), (0,))) + b1, 0.0)
        m = hs0 / E
        v = hs1 / E - m * m
        hn = (h - m) * (gamma / jnp.sqrt(v + 1e-5)) + beta
        return _dot(hn, w2, ((1,), (0,))) + b2

    o1 = tower(ei[:, :FIN], hs_ref[0:1, :], hs_ref[1:2, :])
    o2 = tower(ei[:, FIN:], hs_ref[2:3, :], hs_ref[3:4, :])
    n1 = jnp.maximum(jnp.sqrt(jnp.sum(o1 * o1, axis=1, keepdims=True)), 1e-8)
    n2 = jnp.maximum(jnp.sqrt(jnp.sum(o2 * o2, axis=1, keepdims=True)), 1e-8)
    cos = jnp.sum(o1 * o2, axis=1, keepdims=True) / (n1 * n2)
    w_ref[...] = ((cos + 1.0) * 0.5)[None]

    i = pl.program_id(0)
    eb = ei_ref[0, :, pl.ds(i * EB3, EB3)]
    src = eb[0:1, :]
    dst = eb[1:2, :]
    fsd_ref[...] = (src * N + dst)[:, :, None]
    fds_ref[...] = (dst * N + src)[:, :, None]


def _edgeout(edge_input, stats, hstats, W1, b1, gamma, beta, W2, b2, eiT):
    c = pl.pallas_call(
        _edgeout_body,
        grid=(ES3,),
        in_specs=[
            pl.BlockSpec((EB3, D), lambda i: (i, 0)),
            pl.BlockSpec((2, D), lambda i: (0, 0)),
            pl.BlockSpec((4, D), lambda i: (0, 0)),
            pl.BlockSpec((FIN, H), lambda i: (0, 0)),
            pl.BlockSpec((1, H), lambda i: (0, 0)),
            pl.BlockSpec((1, H), lambda i: (0, 0)),
            pl.BlockSpec((1, H), lambda i: (0, 0)),
            pl.BlockSpec((H, H), lambda i: (0, 0)),
            pl.BlockSpec((1, H), lambda i: (0, 0)),
            pl.BlockSpec((1, 2, E), lambda i: (0, 0, 0)),
        ],
        out_specs=[
            pl.BlockSpec((1, EB3, 1), lambda i: (i, 0, 0)),
            pl.BlockSpec((1, EB3, 1), lambda i: (i, 0, 0)),
            pl.BlockSpec((1, EB3, 1), lambda i: (i, 0, 0)),
        ],
        out_shape=[
            jax.ShapeDtypeStruct((ES3, EB3, 1), jnp.float32),
            jax.ShapeDtypeStruct((ES3, EB3, 1), jnp.int32),
            jax.ShapeDtypeStruct((ES3, EB3, 1), jnp.int32),
        ],
    )
    return c(edge_input, stats, hstats, W1, b1, gamma, beta, W2, b2, eiT)


def _topk_body(e_ref, t_ref, a_ref, at_ref, normed):
    m = pl.program_id(0)
    i = pl.program_id(1)

    @pl.when(i == 0)
    def _():
        x = jnp.where(m == 0, e_ref[...], t_ref[...])
        normed[...] = x / jnp.sqrt(jnp.sum(x * x, axis=1, keepdims=True))

    nb = normed[pl.ds(i * RB, RB), :]
    s = _dot(nb, normed[...], ((1,), (1,)))
    s = (s + 1.0) * 0.5
    colidx = lax.broadcasted_iota(jnp.int32, (RB, N), 1)
    for _ in range(KNN):
        mx = jnp.max(s, axis=1, keepdims=True)
        cand = jnp.min(jnp.where(s >= mx, colidx, N), axis=1, keepdims=True)
        s = jnp.where(colidx == cand, -jnp.inf, s)
    ablk = jnp.where(s == -jnp.inf, 1.0, 0.0)
    a_ref[0] = ablk
    at_ref[0] = ablk.T


def _topk(emb, t1):
    return pl.pallas_call(
        _topk_body,
        grid=(2, NRB),
        in_specs=[pl.BlockSpec((N, D), lambda m, i: (0, 0)),
                  pl.BlockSpec((N, D), lambda m, i: (0, 0))],
        out_specs=[
            pl.BlockSpec((1, RB, N), lambda m, i: (m, i, 0)),
            pl.BlockSpec((1, N, RB), lambda m, i: (m, 0, i)),
        ],
        out_shape=[
            jax.ShapeDtypeStruct((2, N, N), jnp.float32),
            jax.ShapeDtypeStruct((2, N, N), jnp.float32),
        ],
        scratch_shapes=[pltpu.VMEM((N, D), jnp.float32)],
    )(emb, t1)


def _sc_scatter(fsd_t, fds_t, val_t):
    mesh = plsc.VectorSubcoreMesh(core_axis_name="c", subcore_axis_name="s")

    @functools.partial(
        pl.kernel,
        out_type=jax.ShapeDtypeStruct((NN,), jnp.float32),
        mesh=mesh,
        scratch_types=[
            pltpu.VMEM((2 * SC_CH, 128), jnp.int32),
            pltpu.VMEM((SC_CH, 128), jnp.float32),
            pltpu.VMEM((2 * SC_CH, 128), jnp.int32),
            pltpu.VMEM((ZW,), jnp.float32),
            pltpu.VMEM_SHARED((SPW,), jnp.float32),
            pltpu.SemaphoreType.DMA,
            pltpu.SemaphoreType.DMA,
        ],
    )
    def scat(fsd_hbm, fds_hbm, val_hbm, out_hbm, idx_v, val_v, sidx_v, zbuf,
             sp, sem, zsem):
        cid = lax.axis_index("c")
        sid = lax.axis_index("s")

        pltpu.sync_copy(fsd_hbm.at[sid], idx_v.at[pl.ds(0, SC_CH)])
        pltpu.sync_copy(fds_hbm.at[sid], idx_v.at[pl.ds(SC_CH, SC_CH)])
        pltpu.sync_copy(val_hbm.at[sid], val_v)

        def zinit(j, carry):
            zbuf[pl.ds(j * 16, 16)] = jnp.zeros((16,), jnp.float32)
            return carry

        lax.fori_loop(0, ZW // 16, zinit, 0)

        stripe = QW // 16
        tbase = QW + sid * (2 * SC_CH * 128)

        for p in range(2):
            base = (2 * cid + p) * QW

            for j in range(stripe // ZW):
                pltpu.async_copy(
                    zbuf, sp.at[pl.ds(sid * stripe + j * ZW, ZW)], zsem)

            def clamp(ci, carry):
                for l in range(8):
                    v = idx_v[ci, pl.ds(l * 16, 16)] - base
                    keep = (v >= 0) & (v < QW)
                    trash = (tbase + ci * 128 + l * 16) + lax.iota(
                        jnp.int32, 16)
                    sidx_v[ci, pl.ds(l * 16, 16)] = jnp.where(keep, v, trash)
                return carry

            lax.fori_loop(0, 2 * SC_CH, clamp, 0)

            for j in range(stripe // ZW):
                pltpu.make_async_copy(
                    zbuf, sp.at[pl.ds(sid * stripe + j * ZW, ZW)],
                    zsem).wait()
            plsc.subcore_barrier()

            def sloop(g, carry):
                cps = []
                for b in range(SC_G):
                    ci = g * SC_G + b
                    cps.append(pltpu.async_copy(
                        val_v.at[ci - SC_CH * (ci // SC_CH)],
                        sp.at[sidx_v.at[ci]], sem))
                for cp in cps:
                    cp.wait()
                return carry

            lax.fori_loop(0, 2 * SC_CH // SC_G, sloop, 0)
            plsc.subcore_barrier()

            for j in range(stripe // ZW):
                pltpu.async_copy(
                    sp.at[pl.ds(sid * stripe + j * ZW, ZW)],
                    out_hbm.at[pl.ds(base + sid * stripe + j * ZW, ZW)],
                    zsem)
            for j in range(stripe // ZW):
                pltpu.make_async_copy(
                    sp.at[pl.ds(sid * stripe + j * ZW, ZW)],
                    out_hbm.at[pl.ds(base + sid * stripe + j * ZW, ZW)],
                    zsem).wait()

    return scat(fsd_t, fds_t, val_t)


def _rownorm_body(sadj_ref, a_ref, at_ref, o_ref):
    m = pl.program_id(0)
    i = pl.program_id(1)
    knn = jnp.maximum(a_ref[0], at_ref[0])
    a = jnp.where(m == 0, sadj_ref[0], knn)
    rowg = i * RB + lax.broadcasted_iota(jnp.int32, (RB, 1), 0)
    col = lax.broadcasted_iota(jnp.int32, (RB, N), 1)
    a = a + jnp.where(col == rowg, 1.0, 0.0)
    s = jnp.sum(a, axis=1, keepdims=True)
    o_ref[0] = a * (1.0 / s)


def _rownorm(sadj17, knn_a, knn_at):
    return pl.pallas_call(
        _rownorm_body,
        grid=(3, NRB),
        in_specs=[
            pl.BlockSpec((1, RB, N),
                         lambda m, i: ((m == 0) * i, 0, 0)),
            pl.BlockSpec((1, RB, N),
                         lambda m, i: ((m > 0) * (m - 1), (m > 0) * i, 0)),
            pl.BlockSpec((1, RB, N),
                         lambda m, i: ((m > 0) * (m - 1), (m > 0) * i, 0)),
        ],
        out_specs=pl.BlockSpec((1, RB, N), lambda m, i: (m, i, 0)),
        out_shape=jax.ShapeDtypeStruct((3, N, N), jnp.float32),
    )(sadj17, knn_a, knn_at)


def kernel(embeddings, t1_features, edge_input, W1, b1, gamma, beta, W2, b2,
           edge_index):
    b1r = b1.reshape(1, H)
    gr = gamma.reshape(1, H)
    ber = beta.reshape(1, H)
    b2r = b2.reshape(1, H)
    ei2 = edge_index.astype(jnp.int32).reshape(1, 2, E)

    stats = _estats(edge_input)
    hstats = _hstats(edge_input, stats, W1, b1r)
    w3, fsd3, fds3 = _edgeout(edge_input, stats, hstats, W1, b1r, gr, ber,
                              W2, b2r, ei2)

    fsd_t = fsd3.reshape(16, SC_CH, 128)
    fds_t = fds3.reshape(16, SC_CH, 128)
    val_t = w3.reshape(16, SC_CH, 128)
    sadj_flat = _sc_scatter(fsd_t, fds_t, val_t)

    knn_a, knn_at = _topk(embeddings, t1_features)

    sadj16 = sadj_flat.reshape(NN // (RB * N), RB, N)
    return _rownorm(sadj16, knn_a, knn_at)


# speedup vs baseline: 4.1091x; 1.2511x over previous
import functools

import jax
import jax.numpy as jnp
from jax import lax
from jax.experimental import pallas as pl
from jax.experimental.pallas import tpu as pltpu
from jax.experimental.pallas import tpu_sc as plsc

N = 2048
E = 65536
D = 128
FIN = 64
H = 128
KNN = 10

NN = N * N
QW = NN // 4
TW = 131072
SPW = QW + TW

EB1 = 8192
ES1 = E // EB1
EB3 = 2048
ES3 = E // EB3
RB = 128
NRB = N // RB

SC_TOT = 2 * E
SC_CH = E // 16 // 128
SC_G = 8
ZW = 32768


def _dot(a, b, dims):
    return lax.dot_general(a, b, (dims, ((), ())),
                           precision=lax.Precision.DEFAULT,
                           preferred_element_type=jnp.float32)


def _estats_body(x_ref, o_ref):
    i = pl.program_id(0)
    x = x_ref[...]
    blk = jnp.concatenate([jnp.sum(x, axis=0, keepdims=True),
                           jnp.sum(x * x, axis=0, keepdims=True)], axis=0)

    @pl.when(i == 0)
    def _():
        o_ref[...] = blk

    @pl.when(i > 0)
    def _():
        o_ref[...] += blk


def _estats(edge_input):
    return pl.pallas_call(
        _estats_body,
        grid=(ES1,),
        in_specs=[pl.BlockSpec((EB1, D), lambda i: (i, 0))],
        out_specs=pl.BlockSpec((2, D), lambda i: (0, 0)),
        out_shape=jax.ShapeDtypeStruct((2, D), jnp.float32),
    )(edge_input)


def _standardize(x, st_ref):
    mu = st_ref[0:1, :] / E
    ex2 = st_ref[1:2, :] / E
    sig = jnp.sqrt(jnp.maximum(ex2 - mu * mu, 0.0)) + 1e-6
    return (x - mu) / sig


def _hstats_body(x_ref, st_ref, w1_ref, b1_ref, o_ref):
    i = pl.program_id(0)
    ei = _standardize(x_ref[...], st_ref)
    w1 = w1_ref[...]
    b1 = b1_ref[...]
    h1 = jnp.maximum(_dot(ei[:, :FIN], w1, ((1,), (0,))) + b1, 0.0)
    h2 = jnp.maximum(_dot(ei[:, FIN:], w1, ((1,), (0,))) + b1, 0.0)
    blk = jnp.concatenate([
        jnp.sum(h1, axis=0, keepdims=True),
        jnp.sum(h1 * h1, axis=0, keepdims=True),
        jnp.sum(h2, axis=0, keepdims=True),
        jnp.sum(h2 * h2, axis=0, keepdims=True),
    ], axis=0)

    @pl.when(i == 0)
    def _():
        o_ref[...] = blk

    @pl.when(i > 0)
    def _():
        o_ref[...] += blk


def _hstats(edge_input, stats, W1, b1):
    return pl.pallas_call(
        _hstats_body,
        grid=(ES1,),
        in_specs=[
            pl.BlockSpec((EB1, D), lambda i: (i, 0)),
            pl.BlockSpec((2, D), lambda i: (0, 0)),
            pl.BlockSpec((FIN, H), lambda i: (0, 0)),
            pl.BlockSpec((1, H), lambda i: (0, 0)),
        ],
        out_specs=pl.BlockSpec((4, D), lambda i: (0, 0)),
        out_shape=jax.ShapeDtypeStruct((4, D), jnp.float32),
    )(edge_input, stats, W1, b1)


def _edgeout_body(x_ref, st_ref, hs_ref, w1_ref, b1_ref, g_ref, be_ref,
                  w2_ref, b2_ref, ei_ref, w_ref, fsd_ref, fds_ref):
    ei = _standardize(x_ref[...], st_ref)
    w1 = w1_ref[...]
    b1 = b1_ref[...]
    gamma = g_ref[...]
    beta = be_ref[...]
    w2 = w2_ref[...]
    b2 = b2_ref[...]

    def tower(z, hs0, hs1):
        h = jnp.maximum(_dot(z, w1, ((1,), (0,))) + b1, 0.0)
        m = hs0 / E
        v = hs1 / E - m * m
        hn = (h - m) * (gamma / jnp.sqrt(v + 1e-5)) + beta
        return _dot(hn, w2, ((1,), (0,))) + b2

    o1 = tower(ei[:, :FIN], hs_ref[0:1, :], hs_ref[1:2, :])
    o2 = tower(ei[:, FIN:], hs_ref[2:3, :], hs_ref[3:4, :])
    n1 = jnp.maximum(jnp.sqrt(jnp.sum(o1 * o1, axis=1, keepdims=True)), 1e-8)
    n2 = jnp.maximum(jnp.sqrt(jnp.sum(o2 * o2, axis=1, keepdims=True)), 1e-8)
    cos = jnp.sum(o1 * o2, axis=1, keepdims=True) / (n1 * n2)
    w_ref[...] = ((cos + 1.0) * 0.5)[None]

    i = pl.program_id(0)
    eb = ei_ref[0, :, pl.ds(i * EB3, EB3)]
    src = eb[0:1, :]
    dst = eb[1:2, :]
    fsd_ref[...] = (src * N + dst)[:, :, None]
    fds_ref[...] = (dst * N + src)[:, :, None]


def _edgeout(edge_input, stats, hstats, W1, b1, gamma, beta, W2, b2, eiT):
    c = pl.pallas_call(
        _edgeout_body,
        grid=(ES3,),
        in_specs=[
            pl.BlockSpec((EB3, D), lambda i: (i, 0)),
            pl.BlockSpec((2, D), lambda i: (0, 0)),
            pl.BlockSpec((4, D), lambda i: (0, 0)),
            pl.BlockSpec((FIN, H), lambda i: (0, 0)),
            pl.BlockSpec((1, H), lambda i: (0, 0)),
            pl.BlockSpec((1, H), lambda i: (0, 0)),
            pl.BlockSpec((1, H), lambda i: (0, 0)),
            pl.BlockSpec((H, H), lambda i: (0, 0)),
            pl.BlockSpec((1, H), lambda i: (0, 0)),
            pl.BlockSpec((1, 2, E), lambda i: (0, 0, 0)),
        ],
        out_specs=[
            pl.BlockSpec((1, EB3, 1), lambda i: (i, 0, 0)),
            pl.BlockSpec((1, EB3, 1), lambda i: (i, 0, 0)),
            pl.BlockSpec((1, EB3, 1), lambda i: (i, 0, 0)),
        ],
        out_shape=[
            jax.ShapeDtypeStruct((ES3, EB3, 1), jnp.float32),
            jax.ShapeDtypeStruct((ES3, EB3, 1), jnp.int32),
            jax.ShapeDtypeStruct((ES3, EB3, 1), jnp.int32),
        ],
    )
    return c(edge_input, stats, hstats, W1, b1, gamma, beta, W2, b2, eiT)


def _topk_body(e_ref, t_ref, a_ref, at_ref, normed):
    m = pl.program_id(0)
    i = pl.program_id(1)

    @pl.when(i == 0)
    def _():
        x = jnp.where(m == 0, e_ref[...], t_ref[...])
        normed[...] = x / jnp.sqrt(jnp.sum(x * x, axis=1, keepdims=True))

    nb = normed[pl.ds(i * RB, RB), :]
    s = _dot(nb, normed[...], ((1,), (1,)))
    s = (s + 1.0) * 0.5
    for _ in range(KNN):
        mx = jnp.max(s, axis=1, keepdims=True)
        s = jnp.where(s >= mx, -jnp.inf, s)
    ablk = jnp.where(s == -jnp.inf, 1.0, 0.0)
    a_ref[0] = ablk
    at_ref[0] = ablk.T


def _topk(emb, t1):
    return pl.pallas_call(
        _topk_body,
        grid=(2, NRB),
        in_specs=[pl.BlockSpec((N, D), lambda m, i: (0, 0)),
                  pl.BlockSpec((N, D), lambda m, i: (0, 0))],
        out_specs=[
            pl.BlockSpec((1, RB, N), lambda m, i: (m, i, 0)),
            pl.BlockSpec((1, N, RB), lambda m, i: (m, 0, i)),
        ],
        out_shape=[
            jax.ShapeDtypeStruct((2, N, N), jnp.float32),
            jax.ShapeDtypeStruct((2, N, N), jnp.float32),
        ],
        scratch_shapes=[pltpu.VMEM((N, D), jnp.float32)],
    )(emb, t1)


def _sc_scatter(fsd_t, fds_t, val_t):
    mesh = plsc.VectorSubcoreMesh(core_axis_name="c", subcore_axis_name="s")

    @functools.partial(
        pl.kernel,
        out_type=jax.ShapeDtypeStruct((NN,), jnp.float32),
        mesh=mesh,
        scratch_types=[
            pltpu.VMEM((2 * SC_CH, 128), jnp.int32),
            pltpu.VMEM((SC_CH, 128), jnp.float32),
            pltpu.VMEM((2 * SC_CH, 128), jnp.int32),
            pltpu.VMEM((ZW,), jnp.float32),
            pltpu.VMEM_SHARED((SPW,), jnp.float32),
            pltpu.SemaphoreType.DMA,
            pltpu.SemaphoreType.DMA,
        ],
    )
    def scat(fsd_hbm, fds_hbm, val_hbm, out_hbm, idx_v, val_v, sidx_v, zbuf,
             sp, sem, zsem):
        cid = lax.axis_index("c")
        sid = lax.axis_index("s")

        pltpu.sync_copy(fsd_hbm.at[sid], idx_v.at[pl.ds(0, SC_CH)])
        pltpu.sync_copy(fds_hbm.at[sid], idx_v.at[pl.ds(SC_CH, SC_CH)])
        pltpu.sync_copy(val_hbm.at[sid], val_v)

        def zinit(j, carry):
            zbuf[pl.ds(j * 16, 16)] = jnp.zeros((16,), jnp.float32)
            return carry

        lax.fori_loop(0, ZW // 16, zinit, 0)

        stripe = QW // 16
        tbase = QW + sid * (2 * SC_CH * 128)

        for p in range(2):
            base = (2 * cid + p) * QW

            for j in range(stripe // ZW):
                pltpu.async_copy(
                    zbuf, sp.at[pl.ds(sid * stripe + j * ZW, ZW)], zsem)

            def clamp(ci, carry):
                for l in range(8):
                    v = idx_v[ci, pl.ds(l * 16, 16)] - base
                    keep = (v >= 0) & (v < QW)
                    trash = (tbase + ci * 128 + l * 16) + lax.iota(
                        jnp.int32, 16)
                    sidx_v[ci, pl.ds(l * 16, 16)] = jnp.where(keep, v, trash)
                return carry

            lax.fori_loop(0, 2 * SC_CH, clamp, 0)

            for j in range(stripe // ZW):
                pltpu.make_async_copy(
                    zbuf, sp.at[pl.ds(sid * stripe + j * ZW, ZW)],
                    zsem).wait()
            plsc.subcore_barrier()

            def sloop(g, carry):
                cps = []
                for b in range(SC_G):
                    ci = g * SC_G + b
                    cps.append(pltpu.async_copy(
                        val_v.at[ci - SC_CH * (ci // SC_CH)],
                        sp.at[sidx_v.at[ci]], sem))
                for cp in cps:
                    cp.wait()
                return carry

            lax.fori_loop(0, 2 * SC_CH // SC_G, sloop, 0)
            plsc.subcore_barrier()

            for j in range(stripe // ZW):
                pltpu.async_copy(
                    sp.at[pl.ds(sid * stripe + j * ZW, ZW)],
                    out_hbm.at[pl.ds(base + sid * stripe + j * ZW, ZW)],
                    zsem)
            for j in range(stripe // ZW):
                pltpu.make_async_copy(
                    sp.at[pl.ds(sid * stripe + j * ZW, ZW)],
                    out_hbm.at[pl.ds(base + sid * stripe + j * ZW, ZW)],
                    zsem).wait()

    return scat(fsd_t, fds_t, val_t)


def _rownorm_body(sadj_ref, a_ref, at_ref, o_ref):
    m = pl.program_id(0)
    i = pl.program_id(1)
    knn = jnp.maximum(a_ref[0], at_ref[0])
    a = jnp.where(m == 0, sadj_ref[0], knn)
    rowg = i * RB + lax.broadcasted_iota(jnp.int32, (RB, 1), 0)
    col = lax.broadcasted_iota(jnp.int32, (RB, N), 1)
    a = a + jnp.where(col == rowg, 1.0, 0.0)
    s = jnp.sum(a, axis=1, keepdims=True)
    o_ref[0] = a * (1.0 / s)


def _rownorm(sadj17, knn_a, knn_at):
    return pl.pallas_call(
        _rownorm_body,
        grid=(3, NRB),
        in_specs=[
            pl.BlockSpec((1, RB, N),
                         lambda m, i: ((m == 0) * i, 0, 0)),
            pl.BlockSpec((1, RB, N),
                         lambda m, i: ((m > 0) * (m - 1), (m > 0) * i, 0)),
            pl.BlockSpec((1, RB, N),
                         lambda m, i: ((m > 0) * (m - 1), (m > 0) * i, 0)),
        ],
        out_specs=pl.BlockSpec((1, RB, N), lambda m, i: (m, i, 0)),
        out_shape=jax.ShapeDtypeStruct((3, N, N), jnp.float32),
    )(sadj17, knn_a, knn_at)


def kernel(embeddings, t1_features, edge_input, W1, b1, gamma, beta, W2, b2,
           edge_index):
    b1r = b1.reshape(1, H)
    gr = gamma.reshape(1, H)
    ber = beta.reshape(1, H)
    b2r = b2.reshape(1, H)
    ei2 = edge_index.astype(jnp.int32).reshape(1, 2, E)

    stats = _estats(edge_input)
    hstats = _hstats(edge_input, stats, W1, b1r)
    w3, fsd3, fds3 = _edgeout(edge_input, stats, hstats, W1, b1r, gr, ber,
                              W2, b2r, ei2)

    fsd_t = fsd3.reshape(16, SC_CH, 128)
    fds_t = fds3.reshape(16, SC_CH, 128)
    val_t = w3.reshape(16, SC_CH, 128)
    sadj_flat = _sc_scatter(fsd_t, fds_t, val_t)

    knn_a, knn_at = _topk(embeddings, t1_features)

    sadj16 = sadj_flat.reshape(NN // (RB * N), RB, N)
    return _rownorm(sadj16, knn_a, knn_at)


# speedup vs baseline: 4.1970x; 1.0214x over previous
import functools

import jax
import jax.numpy as jnp
from jax import lax
from jax.experimental import pallas as pl
from jax.experimental.pallas import tpu as pltpu
from jax.experimental.pallas import tpu_sc as plsc

N = 2048
E = 65536
D = 128
FIN = 64
H = 128
KNN = 10

NN = N * N
QW = NN // 4
TW = 131072
SPW = QW + TW

EB1 = 8192
ES1 = E // EB1
EB3 = 2048
ES3 = E // EB3
RB = 128
NRB = N // RB

SC_TOT = 2 * E
SC_CH = E // 16 // 128
SC_G = 8
ZW = 32768


def _dot(a, b, dims):
    return lax.dot_general(a, b, (dims, ((), ())),
                           precision=lax.Precision.DEFAULT,
                           preferred_element_type=jnp.float32)


def _estats_body(x_ref, o_ref):
    i = pl.program_id(0)
    x = x_ref[...]
    blk = jnp.concatenate([jnp.sum(x, axis=0, keepdims=True),
                           jnp.sum(x * x, axis=0, keepdims=True)], axis=0)

    @pl.when(i == 0)
    def _():
        o_ref[...] = blk

    @pl.when(i > 0)
    def _():
        o_ref[...] += blk


def _estats(edge_input):
    return pl.pallas_call(
        _estats_body,
        grid=(ES1,),
        in_specs=[pl.BlockSpec((EB1, D), lambda i: (i, 0))],
        out_specs=pl.BlockSpec((2, D), lambda i: (0, 0)),
        out_shape=jax.ShapeDtypeStruct((2, D), jnp.float32),
    )(edge_input)


def _standardize(x, st_ref):
    mu = st_ref[0:1, :] / E
    ex2 = st_ref[1:2, :] / E
    sig = jnp.sqrt(jnp.maximum(ex2 - mu * mu, 0.0)) + 1e-6
    return (x - mu) / sig


def _hstats_body(x_ref, st_ref, w1_ref, b1_ref, o_ref):
    i = pl.program_id(0)
    ei = _standardize(x_ref[...], st_ref)
    w1 = w1_ref[...]
    b1 = b1_ref[...]
    h1 = jnp.maximum(_dot(ei[:, :FIN], w1, ((1,), (0,))) + b1, 0.0)
    h2 = jnp.maximum(_dot(ei[:, FIN:], w1, ((1,), (0,))) + b1, 0.0)
    blk = jnp.concatenate([
        jnp.sum(h1, axis=0, keepdims=True),
        jnp.sum(h1 * h1, axis=0, keepdims=True),
        jnp.sum(h2, axis=0, keepdims=True),
        jnp.sum(h2 * h2, axis=0, keepdims=True),
    ], axis=0)

    @pl.when(i == 0)
    def _():
        o_ref[...] = blk

    @pl.when(i > 0)
    def _():
        o_ref[...] += blk


def _hstats(edge_input, stats, W1, b1):
    return pl.pallas_call(
        _hstats_body,
        grid=(ES1,),
        in_specs=[
            pl.BlockSpec((EB1, D), lambda i: (i, 0)),
            pl.BlockSpec((2, D), lambda i: (0, 0)),
            pl.BlockSpec((FIN, H), lambda i: (0, 0)),
            pl.BlockSpec((1, H), lambda i: (0, 0)),
        ],
        out_specs=pl.BlockSpec((4, D), lambda i: (0, 0)),
        out_shape=jax.ShapeDtypeStruct((4, D), jnp.float32),
    )(edge_input, stats, W1, b1)


def _edgeout_body(x_ref, st_ref, hs_ref, w1_ref, b1_ref, g_ref, be_ref,
                  w2_ref, b2_ref, ei_ref, w_ref, fsd_ref, fds_ref):
    ei = _standardize(x_ref[...], st_ref)
    w1 = w1_ref[...]
    b1 = b1_ref[...]
    gamma = g_ref[...]
    beta = be_ref[...]
    w2 = w2_ref[...]
    b2 = b2_ref[...]

    def tower(z, hs0, hs1):
        h = jnp.maximum(_dot(z, w1, ((1,), (0,))) + b1, 0.0)
        m = hs0 / E
        v = hs1 / E - m * m
        hn = (h - m) * (gamma / jnp.sqrt(v + 1e-5)) + beta
        return _dot(hn, w2, ((1,), (0,))) + b2

    o1 = tower(ei[:, :FIN], hs_ref[0:1, :], hs_ref[1:2, :])
    o2 = tower(ei[:, FIN:], hs_ref[2:3, :], hs_ref[3:4, :])
    n1 = jnp.maximum(jnp.sqrt(jnp.sum(o1 * o1, axis=1, keepdims=True)), 1e-8)
    n2 = jnp.maximum(jnp.sqrt(jnp.sum(o2 * o2, axis=1, keepdims=True)), 1e-8)
    cos = jnp.sum(o1 * o2, axis=1, keepdims=True) / (n1 * n2)
    w_ref[...] = ((cos + 1.0) * 0.5)[None]

    i = pl.program_id(0)
    eb = ei_ref[0, :, pl.ds(i * EB3, EB3)]
    src = eb[0:1, :]
    dst = eb[1:2, :]
    fsd_ref[...] = (src * N + dst)[:, :, None]
    fds_ref[...] = (dst * N + src)[:, :, None]


def _edgeout(edge_input, stats, hstats, W1, b1, gamma, beta, W2, b2, eiT):
    c = pl.pallas_call(
        _edgeout_body,
        grid=(ES3,),
        in_specs=[
            pl.BlockSpec((EB3, D), lambda i: (i, 0)),
            pl.BlockSpec((2, D), lambda i: (0, 0)),
            pl.BlockSpec((4, D), lambda i: (0, 0)),
            pl.BlockSpec((FIN, H), lambda i: (0, 0)),
            pl.BlockSpec((1, H), lambda i: (0, 0)),
            pl.BlockSpec((1, H), lambda i: (0, 0)),
            pl.BlockSpec((1, H), lambda i: (0, 0)),
            pl.BlockSpec((H, H), lambda i: (0, 0)),
            pl.BlockSpec((1, H), lambda i: (0, 0)),
            pl.BlockSpec((1, 2, E), lambda i: (0, 0, 0)),
        ],
        out_specs=[
            pl.BlockSpec((1, EB3, 1), lambda i: (i, 0, 0)),
            pl.BlockSpec((1, EB3, 1), lambda i: (i, 0, 0)),
            pl.BlockSpec((1, EB3, 1), lambda i: (i, 0, 0)),
        ],
        out_shape=[
            jax.ShapeDtypeStruct((ES3, EB3, 1), jnp.float32),
            jax.ShapeDtypeStruct((ES3, EB3, 1), jnp.int32),
            jax.ShapeDtypeStruct((ES3, EB3, 1), jnp.int32),
        ],
    )
    return c(edge_input, stats, hstats, W1, b1, gamma, beta, W2, b2, eiT)


def _topk_body(e_ref, t_ref, a_ref, at_ref, normed):
    m = pl.program_id(0)
    i = pl.program_id(1)

    @pl.when(i == 0)
    def _():
        x = jnp.where(m == 0, e_ref[...], t_ref[...])
        normed[...] = x / jnp.sqrt(jnp.sum(x * x, axis=1, keepdims=True))

    nb = normed[pl.ds(i * RB, RB), :]
    s = _dot(nb, normed[...], ((1,), (1,)))
    s = (s + 1.0) * 0.5
    for _ in range(KNN):
        mx = jnp.max(s, axis=1, keepdims=True)
        s = jnp.where(s >= mx, -jnp.inf, s)
    ablk = jnp.where(s == -jnp.inf, 1.0, 0.0).astype(jnp.bfloat16)
    a_ref[0] = ablk
    at_ref[0] = ablk.T


def _topk(emb, t1):
    return pl.pallas_call(
        _topk_body,
        grid=(2, NRB),
        in_specs=[pl.BlockSpec((N, D), lambda m, i: (0, 0)),
                  pl.BlockSpec((N, D), lambda m, i: (0, 0))],
        out_specs=[
            pl.BlockSpec((1, RB, N), lambda m, i: (m, i, 0)),
            pl.BlockSpec((1, N, RB), lambda m, i: (m, 0, i)),
        ],
        out_shape=[
            jax.ShapeDtypeStruct((2, N, N), jnp.bfloat16),
            jax.ShapeDtypeStruct((2, N, N), jnp.bfloat16),
        ],
        scratch_shapes=[pltpu.VMEM((N, D), jnp.float32)],
    )(emb, t1)


def _sc_scatter(fsd_t, fds_t, val_t):
    mesh = plsc.VectorSubcoreMesh(core_axis_name="c", subcore_axis_name="s")

    @functools.partial(
        pl.kernel,
        out_type=jax.ShapeDtypeStruct((NN,), jnp.float32),
        mesh=mesh,
        scratch_types=[
            pltpu.VMEM((2 * SC_CH, 128), jnp.int32),
            pltpu.VMEM((SC_CH, 128), jnp.float32),
            pltpu.VMEM((2 * SC_CH, 128), jnp.int32),
            pltpu.VMEM((ZW,), jnp.float32),
            pltpu.VMEM_SHARED((SPW,), jnp.float32),
            pltpu.SemaphoreType.DMA,
            pltpu.SemaphoreType.DMA,
        ],
    )
    def scat(fsd_hbm, fds_hbm, val_hbm, out_hbm, idx_v, val_v, sidx_v, zbuf,
             sp, sem, zsem):
        cid = lax.axis_index("c")
        sid = lax.axis_index("s")

        pltpu.sync_copy(fsd_hbm.at[sid], idx_v.at[pl.ds(0, SC_CH)])
        pltpu.sync_copy(fds_hbm.at[sid], idx_v.at[pl.ds(SC_CH, SC_CH)])
        pltpu.sync_copy(val_hbm.at[sid], val_v)

        def zinit(j, carry):
            zbuf[pl.ds(j * 16, 16)] = jnp.zeros((16,), jnp.float32)
            return carry

        lax.fori_loop(0, ZW // 16, zinit, 0)

        stripe = QW // 16
        tbase = QW + sid * (2 * SC_CH * 128)

        for p in range(2):
            base = (2 * cid + p) * QW

            for j in range(stripe // ZW):
                pltpu.async_copy(
                    zbuf, sp.at[pl.ds(sid * stripe + j * ZW, ZW)], zsem)

            def clamp(ci, carry):
                for l in range(8):
                    v = idx_v[ci, pl.ds(l * 16, 16)] - base
                    keep = (v >= 0) & (v < QW)
                    trash = (tbase + ci * 128 + l * 16) + lax.iota(
                        jnp.int32, 16)
                    sidx_v[ci, pl.ds(l * 16, 16)] = jnp.where(keep, v, trash)
                return carry

            lax.fori_loop(0, 2 * SC_CH, clamp, 0)

            for j in range(stripe // ZW):
                pltpu.make_async_copy(
                    zbuf, sp.at[pl.ds(sid * stripe + j * ZW, ZW)],
                    zsem).wait()
            plsc.subcore_barrier()

            def sloop(g, carry):
                cps = []
                for b in range(SC_G):
                    ci = g * SC_G + b
                    cps.append(pltpu.async_copy(
                        val_v.at[ci - SC_CH * (ci // SC_CH)],
                        sp.at[sidx_v.at[ci]], sem))
                for cp in cps:
                    cp.wait()
                return carry

            lax.fori_loop(0, 2 * SC_CH // SC_G, sloop, 0)
            plsc.subcore_barrier()

            for j in range(stripe // ZW):
                pltpu.async_copy(
                    sp.at[pl.ds(sid * stripe + j * ZW, ZW)],
                    out_hbm.at[pl.ds(base + sid * stripe + j * ZW, ZW)],
                    zsem)
            for j in range(stripe // ZW):
                pltpu.make_async_copy(
                    sp.at[pl.ds(sid * stripe + j * ZW, ZW)],
                    out_hbm.at[pl.ds(base + sid * stripe + j * ZW, ZW)],
                    zsem).wait()

    return scat(fsd_t, fds_t, val_t)


def _rownorm_body(sadj_ref, a_ref, at_ref, o_ref):
    m = pl.program_id(0)
    i = pl.program_id(1)
    knn = jnp.maximum(a_ref[0], at_ref[0]).astype(jnp.float32)
    a = jnp.where(m == 0, sadj_ref[0], knn)
    rowg = i * RB + lax.broadcasted_iota(jnp.int32, (RB, 1), 0)
    col = lax.broadcasted_iota(jnp.int32, (RB, N), 1)
    a = a + jnp.where(col == rowg, 1.0, 0.0)
    s = jnp.sum(a, axis=1, keepdims=True)
    o_ref[0] = a * (1.0 / s)


def _rownorm(sadj17, knn_a, knn_at):
    return pl.pallas_call(
        _rownorm_body,
        grid=(3, NRB),
        in_specs=[
            pl.BlockSpec((1, RB, N),
                         lambda m, i: ((m == 0) * i, 0, 0)),
            pl.BlockSpec((1, RB, N),
                         lambda m, i: ((m > 0) * (m - 1), (m > 0) * i, 0)),
            pl.BlockSpec((1, RB, N),
                         lambda m, i: ((m > 0) * (m - 1), (m > 0) * i, 0)),
        ],
        out_specs=pl.BlockSpec((1, RB, N), lambda m, i: (m, i, 0)),
        out_shape=jax.ShapeDtypeStruct((3, N, N), jnp.float32),
    )(sadj17, knn_a, knn_at)


def kernel(embeddings, t1_features, edge_input, W1, b1, gamma, beta, W2, b2,
           edge_index):
    b1r = b1.reshape(1, H)
    gr = gamma.reshape(1, H)
    ber = beta.reshape(1, H)
    b2r = b2.reshape(1, H)
    ei2 = edge_index.astype(jnp.int32).reshape(1, 2, E)

    stats = _estats(edge_input)
    hstats = _hstats(edge_input, stats, W1, b1r)
    w3, fsd3, fds3 = _edgeout(edge_input, stats, hstats, W1, b1r, gr, ber,
                              W2, b2r, ei2)

    fsd_t = fsd3.reshape(16, SC_CH, 128)
    fds_t = fds3.reshape(16, SC_CH, 128)
    val_t = w3.reshape(16, SC_CH, 128)
    sadj_flat = _sc_scatter(fsd_t, fds_t, val_t)

    knn_a, knn_at = _topk(embeddings, t1_features)

    sadj16 = sadj_flat.reshape(NN // (RB * N), RB, N)
    return _rownorm(sadj16, knn_a, knn_at)
